# baseline (speedup 1.0000x reference)
import functools

import jax
import jax.numpy as jnp
import numpy as np
from jax import lax
from jax.experimental import pallas as pl
from jax.experimental.pallas import tpu as pltpu
from jax.experimental.pallas import tpu_sc as plsc

B = 4
N = 4096
D = 256
NH = 8
NP = 4
HG = 64
WG = 64
QBA = 256
QBS = 8
SBS = 32
NSUB = SBS // QBS
NWORK = 32
QPW = (B * N) // NWORK
NSLOT = NP * 4
NCOL = NH * NSLOT

_SPLAT_DNUMS = lax.GatherDimensionNumbers(
    offset_dims=(), collapsed_slice_dims=(0,), start_index_map=(0,))


def _splat(vec, s):
    return lax.gather(vec, jnp.full((16, 1), s, jnp.int32), _SPLAT_DNUMS, (1,),
                      mode=lax.GatherScatterMode.PROMISE_IN_BOUNDS)


def _stage_a_body(q_ref, rp_ref, val_ref, wv_ref, bv_ref, wox_ref, box_ref,
                  woy_ref, boy_ref, wat_ref, bat_ref, g_ref,
                  vout_ref, idx_ref, wgt_ref):
    q = q_ref[0]
    vout_ref[0] = (
        jnp.dot(val_ref[0], wv_ref[...], preferred_element_type=jnp.float32)
        + bv_ref[...]
    )
    offx = jnp.dot(q, wox_ref[...], preferred_element_type=jnp.float32) + box_ref[...]
    offy = jnp.dot(q, woy_ref[...], preferred_element_type=jnp.float32) + boy_ref[...]
    a = jnp.dot(q, wat_ref[...], preferred_element_type=jnp.float32) + bat_ref[...]
    m = jnp.max(a, axis=-1, keepdims=True)
    e = jnp.exp(a - m)
    denom = jnp.dot(e, g_ref[...], preferred_element_type=jnp.float32)
    attn = e / denom

    rp = rp_ref[0]
    x = rp[:, 0:1] * float(WG) - 0.5 + offx
    y = rp[:, 1:2] * float(HG) - 0.5 + offy
    x0 = jnp.floor(x)
    y0 = jnp.floor(y)
    lx = x - x0
    ly = y - y0
    x0i = x0.astype(jnp.int32)
    y0i = y0.astype(jnp.int32)
    hcol = lax.broadcasted_iota(jnp.int32, (QBA, NH * NP), 1) // NP
    bofs = pl.program_id(0) * (N * NH)
    wcorn = [(1.0 - lx) * (1.0 - ly), lx * (1.0 - ly),
             (1.0 - lx) * ly, lx * ly]

    idx_c = [None] * 4
    wgt_c = [None] * 4
    for c, (dx, dy) in enumerate(((0, 0), (1, 0), (0, 1), (1, 1))):
        cx = x0i + dx
        cy = y0i + dy
        valid = ((cx >= 0) & (cx < WG) & (cy >= 0) & (cy < HG))
        cell = jnp.clip(cy, 0, HG - 1) * WG + jnp.clip(cx, 0, WG - 1)
        idx_c[c] = bofs + cell * NH + hcol
        wgt_c[c] = wcorn[c] * attn * valid.astype(jnp.float32)

    idx_ref[0] = jnp.concatenate(idx_c, axis=1)
    wgt_ref[0] = jnp.concatenate(wgt_c, axis=1)


def _stage_a(query, ref_pts, value, wv, bv, wox, box, woy, boy, wat, bat, g):
    nb = query.shape[0]
    grid = (nb, N // QBA)
    full = lambda shape: pl.BlockSpec(shape, lambda b, j: (0,) * len(shape))
    blk3 = lambda w: pl.BlockSpec((1, QBA, w), lambda b, j: (b, j, 0))
    return pl.pallas_call(
        _stage_a_body,
        grid=grid,
        in_specs=[
            blk3(D), blk3(2), blk3(D),
            full((D, D)), full((1, D)),
            full((D, 32)), full((1, 32)),
            full((D, 32)), full((1, 32)),
            full((D, 32)), full((1, 32)),
            full((32, 32)),
        ],
        out_specs=[blk3(D), blk3(NCOL), blk3(NCOL)],
        out_shape=[
            jax.ShapeDtypeStruct((nb, N, D), jnp.float32),
            jax.ShapeDtypeStruct((nb, N, NCOL), jnp.int32),
            jax.ShapeDtypeStruct((nb, N, NCOL), jnp.float32),
        ],
    )(query, ref_pts, value, wv, bv, wox, box, woy, boy, wat, bat, g)


def _stage_b_body(s_ref, w_ref, b_ref, o_ref):
    o_ref[0] = (
        jnp.dot(s_ref[0], w_ref[...], preferred_element_type=jnp.float32)
        + b_ref[...]
    )


def _stage_b(samp, w_out, b_out2):
    nb = samp.shape[0]
    return pl.pallas_call(
        _stage_b_body,
        grid=(nb, N // QBA),
        in_specs=[
            pl.BlockSpec((1, QBA, D), lambda b, j: (b, j, 0)),
            pl.BlockSpec((D, D), lambda b, j: (0, 0)),
            pl.BlockSpec((1, D), lambda b, j: (0, 0)),
        ],
        out_specs=pl.BlockSpec((1, QBA, D), lambda b, j: (b, j, 0)),
        out_shape=jax.ShapeDtypeStruct((nb, N, D), jnp.float32),
    )(samp, w_out, b_out2)


def _make_sc_body(qpw):
  def _sc_body(vtab, idxt, wgtt, out, idx_v, wgt_v, rows_v, out_v, sem0, sem1):
    wid = lax.axis_index("s") * 2 + lax.axis_index("c")
    base = wid * qpw
    sems = (sem0, sem1)

    def compute_sub(sub):
        rbuf = rows_v.at[sub % 2]

        def q_body(qq, carry2):
            rq = qq * NCOL
            row = sub * QBS + qq
            for h in range(NH):
                woff = 16 * (h // 4)
                lbase = h * 4 - woff
                acc = [jnp.zeros((16,), jnp.float32) for _ in range(4)]
                for c in range(4):
                    wv = wgt_v[row, pl.ds(c * 32 + woff, 16)]
                    rc = rq + c * 32 + h * 4
                    for p in range(4):
                        ws = _splat(wv, lbase + p)
                        lo = rbuf[rc + p, pl.ds(0, 16)]
                        hi = rbuf[rc + p, pl.ds(16, 16)]
                        k = 2 * (p % 2)
                        acc[k] = acc[k] + ws * lo
                        acc[k + 1] = acc[k + 1] + ws * hi
                out_v[row, pl.ds(h * 32, 16)] = acc[0] + acc[2]
                out_v[row, pl.ds(h * 32 + 16, 16)] = acc[1] + acc[3]
            return carry2

        lax.fori_loop(0, QBS, q_body, 0)

    def super_body(sb, carry):
        q0 = base + sb * SBS
        pltpu.sync_copy(idxt.at[pl.ds(q0 * NCOL, SBS * NCOL)], idx_v)
        pltpu.sync_copy(wgtt.at[pl.ds(q0, SBS), :], wgt_v)

        def fire(sub):
            return pltpu.async_copy(
                vtab.at[idx_v.at[pl.ds(sub * QBS * NCOL, QBS * NCOL)]],
                rows_v.at[sub % 2],
                sems[sub % 2],
            )

        handle = fire(0)
        for sub in range(NSUB):
            nxt = fire(sub + 1) if sub + 1 < NSUB else None
            handle.wait()
            compute_sub(sub)
            handle = nxt

        pltpu.sync_copy(out_v, out.at[pl.ds(q0, SBS), :])
        return carry

    lax.fori_loop(0, qpw // SBS, super_body, 0)
  return _sc_body


def _make_sc_gather(bh):
  return functools.partial(
    pl.kernel,
    out_type=jax.ShapeDtypeStruct((bh * N, D), jnp.float32),
    mesh=plsc.VectorSubcoreMesh(core_axis_name="c", subcore_axis_name="s",
                                num_cores=2, num_subcores=16),
    compiler_params=pltpu.CompilerParams(needs_layout_passes=False,
                                         use_tc_tiling_on_sc=False),
    scratch_types=[
        pltpu.VMEM((SBS * NCOL,), jnp.int32),
        pltpu.VMEM((SBS, NCOL), jnp.float32),
        pltpu.VMEM((2, QBS * NCOL, 32), jnp.float32),
        pltpu.VMEM((SBS, D), jnp.float32),
        pltpu.SemaphoreType.DMA,
        pltpu.SemaphoreType.DMA,
    ],
  )(_make_sc_body(bh * N // NWORK))


_sc_gather_halves = (_make_sc_gather(2), _make_sc_gather(2))


def kernel(query, reference_points, value, spatial_shapes, level_start_index,
           W_value, b_value, W_off, b_off, W_attn, b_attn, W_out, b_out):
    wr = W_off.reshape(D, NH * NP, 2)
    wox = wr[..., 0]
    woy = wr[..., 1]
    br = b_off.reshape(NH * NP, 2)
    box = br[..., 0].reshape(1, NH * NP)
    boy = br[..., 1].reshape(1, NH * NP)
    gsum = jnp.asarray(np.kron(np.eye(NH, dtype=np.float32),
                               np.ones((NP, NP), np.float32)))

    halves = []
    for i, sc_g in enumerate(_sc_gather_halves):
        sl = slice(2 * i, 2 * i + 2)
        vout, idxm, wgtm = _stage_a(
            query[sl], reference_points[sl], value[sl],
            W_value, b_value.reshape(1, D),
            wox, box, woy, boy, W_attn, b_attn.reshape(1, NH * NP),
            gsum,
        )
        samp = sc_g(
            vout.reshape(2 * N * NH, D // NH),
            idxm.reshape(2 * N * NCOL),
            wgtm.reshape(2 * N, NCOL),
        )
        halves.append(_stage_b(samp.reshape(2, N, D), W_out,
                               b_out.reshape(1, D)))
    return jnp.concatenate(halves, axis=0)


# speedup vs baseline: 1.1691x; 1.1691x over previous
import functools

import jax
import jax.numpy as jnp
import numpy as np
from jax import lax
from jax.experimental import pallas as pl
from jax.experimental.pallas import tpu as pltpu
from jax.experimental.pallas import tpu_sc as plsc

B = 4
N = 4096
D = 256
NH = 8
NP = 4
HG = 64
WG = 64
QBA = 256
QBS = 8
SBS = 32
NSUB = SBS // QBS
NWORK = 32
QPW = (B * N) // NWORK
NSLOT = NP * 4
NCOL = NH * NSLOT

_SPLAT_DNUMS = lax.GatherDimensionNumbers(
    offset_dims=(), collapsed_slice_dims=(0,), start_index_map=(0,))


def _splat(vec, s):
    return lax.gather(vec, jnp.full((16, 1), s, jnp.int32), _SPLAT_DNUMS, (1,),
                      mode=lax.GatherScatterMode.PROMISE_IN_BOUNDS)


def _stage_a_body(q_ref, rp_ref, val_ref, wv_ref, bv_ref, wox_ref, box_ref,
                  woy_ref, boy_ref, wat_ref, bat_ref, g_ref,
                  vout_ref, idx_ref, wgt_ref):
    q = q_ref[0]
    vout_ref[0] = (
        jnp.dot(val_ref[0], wv_ref[...], preferred_element_type=jnp.float32)
        + bv_ref[...]
    )
    offx = jnp.dot(q, wox_ref[...], preferred_element_type=jnp.float32) + box_ref[...]
    offy = jnp.dot(q, woy_ref[...], preferred_element_type=jnp.float32) + boy_ref[...]
    a = jnp.dot(q, wat_ref[...], preferred_element_type=jnp.float32) + bat_ref[...]
    m = jnp.max(a, axis=-1, keepdims=True)
    e = jnp.exp(a - m)
    denom = jnp.dot(e, g_ref[...], preferred_element_type=jnp.float32)
    attn = e / denom

    rp = rp_ref[0]
    x = rp[:, 0:1] * float(WG) - 0.5 + offx
    y = rp[:, 1:2] * float(HG) - 0.5 + offy
    x0 = jnp.floor(x)
    y0 = jnp.floor(y)
    lx = x - x0
    ly = y - y0
    x0i = x0.astype(jnp.int32)
    y0i = y0.astype(jnp.int32)
    hcol = lax.broadcasted_iota(jnp.int32, (QBA, NH * NP), 1) // NP
    bofs = pl.program_id(0) * (N * NH)
    wcorn = [(1.0 - lx) * (1.0 - ly), lx * (1.0 - ly),
             (1.0 - lx) * ly, lx * ly]

    idx_c = [None] * 4
    wgt_c = [None] * 4
    for c, (dx, dy) in enumerate(((0, 0), (1, 0), (0, 1), (1, 1))):
        cx = x0i + dx
        cy = y0i + dy
        valid = ((cx >= 0) & (cx < WG) & (cy >= 0) & (cy < HG))
        cell = jnp.clip(cy, 0, HG - 1) * WG + jnp.clip(cx, 0, WG - 1)
        idx_c[c] = bofs + cell * NH + hcol
        wgt_c[c] = wcorn[c] * attn * valid.astype(jnp.float32)

    idx_ref[0] = jnp.concatenate(idx_c, axis=1)
    wgt_ref[0] = jnp.concatenate(wgt_c, axis=1)


def _stage_a(query, ref_pts, value, wv, bv, wox, box, woy, boy, wat, bat, g):
    grid = (B, N // QBA)
    full = lambda shape: pl.BlockSpec(shape, lambda b, j: (0,) * len(shape))
    blk3 = lambda w: pl.BlockSpec((1, QBA, w), lambda b, j: (b, j, 0))
    return pl.pallas_call(
        _stage_a_body,
        grid=grid,
        in_specs=[
            blk3(D), blk3(2), blk3(D),
            full((D, D)), full((1, D)),
            full((D, 32)), full((1, 32)),
            full((D, 32)), full((1, 32)),
            full((D, 32)), full((1, 32)),
            full((32, 32)),
        ],
        out_specs=[blk3(D), blk3(NCOL), blk3(NCOL)],
        out_shape=[
            jax.ShapeDtypeStruct((B, N, D), jnp.float32),
            jax.ShapeDtypeStruct((B, N, NCOL), jnp.int32),
            jax.ShapeDtypeStruct((B, N, NCOL), jnp.float32),
        ],
    )(query, ref_pts, value, wv, bv, wox, box, woy, boy, wat, bat, g)


def _stage_b_body(s_ref, w_ref, b_ref, o_ref):
    o_ref[0] = (
        jnp.dot(s_ref[0], w_ref[...], preferred_element_type=jnp.float32)
        + b_ref[...]
    )


def _stage_b(samp, w_out, b_out2):
    return pl.pallas_call(
        _stage_b_body,
        grid=(B, N // QBA),
        in_specs=[
            pl.BlockSpec((1, QBA, D), lambda b, j: (b, j, 0)),
            pl.BlockSpec((D, D), lambda b, j: (0, 0)),
            pl.BlockSpec((1, D), lambda b, j: (0, 0)),
        ],
        out_specs=pl.BlockSpec((1, QBA, D), lambda b, j: (b, j, 0)),
        out_shape=jax.ShapeDtypeStruct((B, N, D), jnp.float32),
    )(samp, w_out, b_out2)


def _sc_body(vtab, idxt, wgtt, out, idx_v, wgt_v, rows_v, out_v, sem0, sem1,
             sema):
    wid = lax.axis_index("s") * 2 + lax.axis_index("c")
    base = wid * QPW
    sems = (sem0, sem1)
    nsb = QPW // SBS

    def prefetch(sb, par):
        q0 = base + sb * SBS
        pltpu.async_copy(idxt.at[pl.ds(q0 * NCOL, SBS * NCOL)],
                         idx_v.at[par], sema)
        pltpu.async_copy(wgtt.at[pl.ds(q0, SBS), :], wgt_v.at[par], sema)

    def drain(par):
        pltpu.make_async_copy(idxt.at[pl.ds(0, SBS * NCOL)],
                              idx_v.at[par], sema).wait()
        pltpu.make_async_copy(wgtt.at[pl.ds(0, SBS), :],
                              wgt_v.at[par], sema).wait()

    def compute_sub(par, sub):
        rbuf = rows_v.at[sub % 2]
        wbuf = wgt_v.at[par]

        def q_body(qq, carry2):
            rq = qq * NCOL
            row = sub * QBS + qq
            wvs = [wbuf[row, pl.ds(k * 16, 16)] for k in range(8)]
            for h in range(NH):
                woff = h // 4
                lbase = (h % 4) * 4
                acc = [jnp.zeros((16,), jnp.float32) for _ in range(4)]
                for c in range(4):
                    wv = wvs[c * 2 + woff]
                    rc = rq + c * 32 + h * 4
                    for p in range(4):
                        ws = _splat(wv, lbase + p)
                        lo = rbuf[rc + p, pl.ds(0, 16)]
                        hi = rbuf[rc + p, pl.ds(16, 16)]
                        k = 2 * (p % 2)
                        acc[k] = acc[k] + ws * lo
                        acc[k + 1] = acc[k + 1] + ws * hi
                out_v[row, pl.ds(h * 32, 16)] = acc[0] + acc[2]
                out_v[row, pl.ds(h * 32 + 16, 16)] = acc[1] + acc[3]
            return carry2

        lax.fori_loop(0, QBS, q_body, 0)

    def super_pair(i, carry):
        for par in (0, 1):
            sb = 2 * i + par
            q0 = base + sb * SBS
            drain(par)

            def fire(sub):
                return pltpu.async_copy(
                    vtab.at[idx_v.at[par].at[pl.ds(sub * QBS * NCOL,
                                                   QBS * NCOL)]],
                    rows_v.at[sub % 2],
                    sems[sub % 2],
                )

            handle = fire(0)
            nxt = sb + 1
            prefetch(jnp.where(nxt < nsb, nxt, 0), 1 - par)
            for sub in range(NSUB):
                nxt_h = fire(sub + 1) if sub + 1 < NSUB else None
                handle.wait()
                compute_sub(par, sub)
                handle = nxt_h

            pltpu.sync_copy(out_v, out.at[pl.ds(q0, SBS), :])
        return carry

    prefetch(0, 0)
    lax.fori_loop(0, QPW // (2 * SBS), super_pair, 0)
    drain(0)


_sc_gather = functools.partial(
    pl.kernel,
    out_type=jax.ShapeDtypeStruct((B * N, D), jnp.float32),
    mesh=plsc.VectorSubcoreMesh(core_axis_name="c", subcore_axis_name="s",
                                num_cores=2, num_subcores=16),
    compiler_params=pltpu.CompilerParams(needs_layout_passes=False,
                                         use_tc_tiling_on_sc=False),
    scratch_types=[
        pltpu.VMEM((2, SBS * NCOL), jnp.int32),
        pltpu.VMEM((2, SBS, NCOL), jnp.float32),
        pltpu.VMEM((2, QBS * NCOL, 32), jnp.float32),
        pltpu.VMEM((SBS, D), jnp.float32),
        pltpu.SemaphoreType.DMA,
        pltpu.SemaphoreType.DMA,
        pltpu.SemaphoreType.DMA,
    ],
)(_sc_body)


def kernel(query, reference_points, value, spatial_shapes, level_start_index,
           W_value, b_value, W_off, b_off, W_attn, b_attn, W_out, b_out):
    wr = W_off.reshape(D, NH * NP, 2)
    wox = wr[..., 0]
    woy = wr[..., 1]
    br = b_off.reshape(NH * NP, 2)
    box = br[..., 0].reshape(1, NH * NP)
    boy = br[..., 1].reshape(1, NH * NP)
    gsum = jnp.asarray(np.kron(np.eye(NH, dtype=np.float32),
                               np.ones((NP, NP), np.float32)))

    vout, idxm, wgtm = _stage_a(
        query, reference_points, value,
        W_value, b_value.reshape(1, D),
        wox, box, woy, boy, W_attn, b_attn.reshape(1, NH * NP),
        gsum,
    )
    samp = _sc_gather(
        vout.reshape(B * N * NH, D // NH),
        idxm.reshape(B * N * NCOL),
        wgtm.reshape(B * N, NCOL),
    )
    return _stage_b(samp.reshape(B, N, D), W_out, b_out.reshape(1, D))


# speedup vs baseline: 1.1697x; 1.0005x over previous
import functools

import jax
import jax.numpy as jnp
import numpy as np
from jax import lax
from jax.experimental import pallas as pl
from jax.experimental.pallas import tpu as pltpu
from jax.experimental.pallas import tpu_sc as plsc

B = 4
N = 4096
D = 256
NH = 8
NP = 4
HG = 64
WG = 64
QBA = 256
QBS = 8
SBS = 32
NSUB = SBS // QBS
NWORK = 32
QPW = (B * N) // NWORK
NSLOT = NP * 4
NCOL = NH * NSLOT

_SPLAT_DNUMS = lax.GatherDimensionNumbers(
    offset_dims=(), collapsed_slice_dims=(0,), start_index_map=(0,))


def _splat(vec, s):
    return lax.gather(vec, jnp.full((16, 1), s, jnp.int32), _SPLAT_DNUMS, (1,),
                      mode=lax.GatherScatterMode.PROMISE_IN_BOUNDS)


def _stage_a_body(q_ref, rp_ref, val_ref, wv_ref, bv_ref, wox_ref, box_ref,
                  woy_ref, boy_ref, wat_ref, bat_ref, g_ref,
                  vout_ref, idx_ref, wgt_ref):
    q = q_ref[0]
    vout_ref[0] = (
        jnp.dot(val_ref[0], wv_ref[...], preferred_element_type=jnp.float32)
        + bv_ref[...]
    )
    offx = jnp.dot(q, wox_ref[...], preferred_element_type=jnp.float32) + box_ref[...]
    offy = jnp.dot(q, woy_ref[...], preferred_element_type=jnp.float32) + boy_ref[...]
    a = jnp.dot(q, wat_ref[...], preferred_element_type=jnp.float32) + bat_ref[...]
    m = jnp.max(a, axis=-1, keepdims=True)
    e = jnp.exp(a - m)
    denom = jnp.dot(e, g_ref[...], preferred_element_type=jnp.float32)
    attn = e / denom

    rp = rp_ref[0]
    x = rp[:, 0:1] * float(WG) - 0.5 + offx
    y = rp[:, 1:2] * float(HG) - 0.5 + offy
    x0 = jnp.floor(x)
    y0 = jnp.floor(y)
    lx = x - x0
    ly = y - y0
    x0i = x0.astype(jnp.int32)
    y0i = y0.astype(jnp.int32)
    hcol = lax.broadcasted_iota(jnp.int32, (QBA, NH * NP), 1) // NP
    bofs = pl.program_id(0) * (N * NH)
    wcorn = [(1.0 - lx) * (1.0 - ly), lx * (1.0 - ly),
             (1.0 - lx) * ly, lx * ly]

    idx_c = [None] * 4
    wgt_c = [None] * 4
    for c, (dx, dy) in enumerate(((0, 0), (1, 0), (0, 1), (1, 1))):
        cx = x0i + dx
        cy = y0i + dy
        valid = ((cx >= 0) & (cx < WG) & (cy >= 0) & (cy < HG))
        cell = jnp.clip(cy, 0, HG - 1) * WG + jnp.clip(cx, 0, WG - 1)
        idx_c[c] = bofs + cell * NH + hcol
        wgt_c[c] = wcorn[c] * attn * valid.astype(jnp.float32)

    idx_ref[0] = jnp.concatenate(idx_c, axis=1)
    wgt_ref[0] = jnp.concatenate(wgt_c, axis=1)


def _stage_a(query, ref_pts, value, wv, bv, wox, box, woy, boy, wat, bat, g):
    grid = (B, N // QBA)
    full = lambda shape: pl.BlockSpec(shape, lambda b, j: (0,) * len(shape))
    blk3 = lambda w: pl.BlockSpec((1, QBA, w), lambda b, j: (b, j, 0))
    return pl.pallas_call(
        _stage_a_body,
        grid=grid,
        in_specs=[
            blk3(D), blk3(2), blk3(D),
            full((D, D)), full((1, D)),
            full((D, 32)), full((1, 32)),
            full((D, 32)), full((1, 32)),
            full((D, 32)), full((1, 32)),
            full((32, 32)),
        ],
        out_specs=[blk3(D), blk3(NCOL), blk3(NCOL)],
        out_shape=[
            jax.ShapeDtypeStruct((B, N, D), jnp.float32),
            jax.ShapeDtypeStruct((B, N, NCOL), jnp.int32),
            jax.ShapeDtypeStruct((B, N, NCOL), jnp.float32),
        ],
    )(query, ref_pts, value, wv, bv, wox, box, woy, boy, wat, bat, g)


def _stage_b_body(s_ref, w_ref, b_ref, o_ref):
    o_ref[0] = (
        jnp.dot(s_ref[0], w_ref[...], preferred_element_type=jnp.float32)
        + b_ref[...]
    )


def _stage_b(samp, w_out, b_out2):
    return pl.pallas_call(
        _stage_b_body,
        grid=(B, N // QBA),
        in_specs=[
            pl.BlockSpec((1, QBA, D), lambda b, j: (b, j, 0)),
            pl.BlockSpec((D, D), lambda b, j: (0, 0)),
            pl.BlockSpec((1, D), lambda b, j: (0, 0)),
        ],
        out_specs=pl.BlockSpec((1, QBA, D), lambda b, j: (b, j, 0)),
        out_shape=jax.ShapeDtypeStruct((B, N, D), jnp.float32),
    )(samp, w_out, b_out2)


def _sc_body(vtab, idxt, wgtt, out, idx_v, wgt_v, rows_v, out_v, sem0, sem1,
             sema, semb):
    wid = lax.axis_index("s") * 2 + lax.axis_index("c")
    base = wid * QPW
    sems = (sem0, sem1)
    nsb = QPW // SBS

    def prefetch(sb, par):
        q0 = base + sb * SBS
        pltpu.async_copy(idxt.at[pl.ds(q0 * NCOL, SBS * NCOL)],
                         idx_v.at[par], sema)
        pltpu.async_copy(wgtt.at[pl.ds(q0, SBS), :], wgt_v.at[par], sema)

    def drain(par):
        pltpu.make_async_copy(idxt.at[pl.ds(0, SBS * NCOL)],
                              idx_v.at[par], sema).wait()
        pltpu.make_async_copy(wgtt.at[pl.ds(0, SBS), :],
                              wgt_v.at[par], sema).wait()

    def compute_sub(par, sub):
        rbuf = rows_v.at[sub % 2]
        wbuf = wgt_v.at[par]
        obuf = out_v.at[par]

        def q_body(qq, carry2):
            rq = qq * NCOL
            row = sub * QBS + qq
            wvs = [wbuf[row, pl.ds(k * 16, 16)] for k in range(8)]
            for h in range(NH):
                woff = h // 4
                lbase = (h % 4) * 4
                acc = [jnp.zeros((16,), jnp.float32) for _ in range(4)]
                for c in range(4):
                    wv = wvs[c * 2 + woff]
                    rc = rq + c * 32 + h * 4
                    for p in range(4):
                        ws = _splat(wv, lbase + p)
                        lo = rbuf[rc + p, pl.ds(0, 16)]
                        hi = rbuf[rc + p, pl.ds(16, 16)]
                        k = 2 * (p % 2)
                        acc[k] = acc[k] + ws * lo
                        acc[k + 1] = acc[k + 1] + ws * hi
                obuf[row, pl.ds(h * 32, 16)] = acc[0] + acc[2]
                obuf[row, pl.ds(h * 32 + 16, 16)] = acc[1] + acc[3]
            return carry2

        lax.fori_loop(0, QBS, q_body, 0)

    def super_pair(i, carry):
        for par in (0, 1):
            sb = 2 * i + par
            q0 = base + sb * SBS
            drain(par)
            @pl.when(i + par > 0)
            def _():
                pltpu.make_async_copy(out_v.at[par],
                                      out.at[pl.ds(base, SBS), :],
                                      semb).wait()

            def fire(sub):
                return pltpu.async_copy(
                    vtab.at[idx_v.at[par].at[pl.ds(sub * QBS * NCOL,
                                                   QBS * NCOL)]],
                    rows_v.at[sub % 2],
                    sems[sub % 2],
                )

            handle = fire(0)
            nxt = sb + 1
            prefetch(jnp.where(nxt < nsb, nxt, 0), 1 - par)
            for sub in range(NSUB):
                nxt_h = fire(sub + 1) if sub + 1 < NSUB else None
                handle.wait()
                compute_sub(par, sub)
                handle = nxt_h

            pltpu.async_copy(out_v.at[par], out.at[pl.ds(q0, SBS), :], semb)
        return carry

    prefetch(0, 0)
    lax.fori_loop(0, QPW // (2 * SBS), super_pair, 0)
    drain(0)
    pltpu.make_async_copy(out_v.at[1], out.at[pl.ds(base, SBS), :],
                          semb).wait()


_sc_gather = functools.partial(
    pl.kernel,
    out_type=jax.ShapeDtypeStruct((B * N, D), jnp.float32),
    mesh=plsc.VectorSubcoreMesh(core_axis_name="c", subcore_axis_name="s",
                                num_cores=2, num_subcores=16),
    compiler_params=pltpu.CompilerParams(needs_layout_passes=False,
                                         use_tc_tiling_on_sc=False),
    scratch_types=[
        pltpu.VMEM((2, SBS * NCOL), jnp.int32),
        pltpu.VMEM((2, SBS, NCOL), jnp.float32),
        pltpu.VMEM((2, QBS * NCOL, 32), jnp.float32),
        pltpu.VMEM((2, SBS, D), jnp.float32),
        pltpu.SemaphoreType.DMA,
        pltpu.SemaphoreType.DMA,
        pltpu.SemaphoreType.DMA,
        pltpu.SemaphoreType.DMA,
    ],
)(_sc_body)


def kernel(query, reference_points, value, spatial_shapes, level_start_index,
           W_value, b_value, W_off, b_off, W_attn, b_attn, W_out, b_out):
    wr = W_off.reshape(D, NH * NP, 2)
    wox = wr[..., 0]
    woy = wr[..., 1]
    br = b_off.reshape(NH * NP, 2)
    box = br[..., 0].reshape(1, NH * NP)
    boy = br[..., 1].reshape(1, NH * NP)
    gsum = jnp.asarray(np.kron(np.eye(NH, dtype=np.float32),
                               np.ones((NP, NP), np.float32)))

    vout, idxm, wgtm = _stage_a(
        query, reference_points, value,
        W_value, b_value.reshape(1, D),
        wox, box, woy, boy, W_attn, b_attn.reshape(1, NH * NP),
        gsum,
    )
    samp = _sc_gather(
        vout.reshape(B * N * NH, D // NH),
        idxm.reshape(B * N * NCOL),
        wgtm.reshape(B * N, NCOL),
    )
    return _stage_b(samp.reshape(B, N, D), W_out, b_out.reshape(1, D))


# speedup vs baseline: 1.2329x; 1.0540x over previous
import functools

import jax
import jax.numpy as jnp
import numpy as np
from jax import lax
from jax.experimental import pallas as pl
from jax.experimental.pallas import tpu as pltpu
from jax.experimental.pallas import tpu_sc as plsc

B = 4
N = 4096
D = 256
NH = 8
NP = 4
HG = 64
WG = 64
QBA = 256
QBS = 8
SBS = 64
NSUB = SBS // QBS
NWORK = 32
QPW = (B * N) // NWORK
NSLOT = NP * 4
NCOL = NH * NSLOT

_SPLAT_DNUMS = lax.GatherDimensionNumbers(
    offset_dims=(), collapsed_slice_dims=(0,), start_index_map=(0,))


def _splat(vec, s):
    return lax.gather(vec, jnp.full((16, 1), s, jnp.int32), _SPLAT_DNUMS, (1,),
                      mode=lax.GatherScatterMode.PROMISE_IN_BOUNDS)


def _stage_a_body(q_ref, rp_ref, val_ref, wv_ref, bv_ref, wox_ref, box_ref,
                  woy_ref, boy_ref, wat_ref, bat_ref, g_ref,
                  vout_ref, idx_ref, wgt_ref):
    q = q_ref[0]
    vout_ref[0] = (
        jnp.dot(val_ref[0], wv_ref[...], preferred_element_type=jnp.float32)
        + bv_ref[...]
    )
    offx = jnp.dot(q, wox_ref[...], preferred_element_type=jnp.float32) + box_ref[...]
    offy = jnp.dot(q, woy_ref[...], preferred_element_type=jnp.float32) + boy_ref[...]
    a = jnp.dot(q, wat_ref[...], preferred_element_type=jnp.float32) + bat_ref[...]
    m = jnp.max(a, axis=-1, keepdims=True)
    e = jnp.exp(a - m)
    denom = jnp.dot(e, g_ref[...], preferred_element_type=jnp.float32)
    attn = e / denom

    rp = rp_ref[0]
    x = rp[:, 0:1] * float(WG) - 0.5 + offx
    y = rp[:, 1:2] * float(HG) - 0.5 + offy
    x0 = jnp.floor(x)
    y0 = jnp.floor(y)
    lx = x - x0
    ly = y - y0
    x0i = x0.astype(jnp.int32)
    y0i = y0.astype(jnp.int32)
    hcol = lax.broadcasted_iota(jnp.int32, (QBA, NH * NP), 1) // NP
    bofs = pl.program_id(0) * (N * NH)
    wcorn = [(1.0 - lx) * (1.0 - ly), lx * (1.0 - ly),
             (1.0 - lx) * ly, lx * ly]

    idx_c = [None] * 4
    wgt_c = [None] * 4
    for c, (dx, dy) in enumerate(((0, 0), (1, 0), (0, 1), (1, 1))):
        cx = x0i + dx
        cy = y0i + dy
        valid = ((cx >= 0) & (cx < WG) & (cy >= 0) & (cy < HG))
        cell = jnp.clip(cy, 0, HG - 1) * WG + jnp.clip(cx, 0, WG - 1)
        idx_c[c] = bofs + cell * NH + hcol
        wgt_c[c] = wcorn[c] * attn * valid.astype(jnp.float32)

    idx_ref[0] = jnp.concatenate(idx_c, axis=1)
    wgt_ref[0] = jnp.concatenate(wgt_c, axis=1)


def _stage_a(query, ref_pts, value, wv, bv, wox, box, woy, boy, wat, bat, g):
    grid = (B, N // QBA)
    full = lambda shape: pl.BlockSpec(shape, lambda b, j: (0,) * len(shape))
    blk3 = lambda w: pl.BlockSpec((1, QBA, w), lambda b, j: (b, j, 0))
    return pl.pallas_call(
        _stage_a_body,
        grid=grid,
        in_specs=[
            blk3(D), blk3(2), blk3(D),
            full((D, D)), full((1, D)),
            full((D, 32)), full((1, 32)),
            full((D, 32)), full((1, 32)),
            full((D, 32)), full((1, 32)),
            full((32, 32)),
        ],
        out_specs=[blk3(D), blk3(NCOL), blk3(NCOL)],
        out_shape=[
            jax.ShapeDtypeStruct((B, N, D), jnp.float32),
            jax.ShapeDtypeStruct((B, N, NCOL), jnp.int32),
            jax.ShapeDtypeStruct((B, N, NCOL), jnp.float32),
        ],
    )(query, ref_pts, value, wv, bv, wox, box, woy, boy, wat, bat, g)


def _stage_b_body(s_ref, w_ref, b_ref, o_ref):
    o_ref[0] = (
        jnp.dot(s_ref[0], w_ref[...], preferred_element_type=jnp.float32)
        + b_ref[...]
    )


def _stage_b(samp, w_out, b_out2):
    return pl.pallas_call(
        _stage_b_body,
        grid=(B, N // QBA),
        in_specs=[
            pl.BlockSpec((1, QBA, D), lambda b, j: (b, j, 0)),
            pl.BlockSpec((D, D), lambda b, j: (0, 0)),
            pl.BlockSpec((1, D), lambda b, j: (0, 0)),
        ],
        out_specs=pl.BlockSpec((1, QBA, D), lambda b, j: (b, j, 0)),
        out_shape=jax.ShapeDtypeStruct((B, N, D), jnp.float32),
    )(samp, w_out, b_out2)


def _sc_body(vtab, idxt, wgtt, out, idx_v, wgt_v, rows_v, out_v, sem0, sem1,
             sema, semb):
    wid = lax.axis_index("s") * 2 + lax.axis_index("c")
    base = wid * QPW
    sems = (sem0, sem1)
    nsb = QPW // SBS

    def prefetch(sb, par):
        q0 = base + sb * SBS
        pltpu.async_copy(idxt.at[pl.ds(q0 * NCOL, SBS * NCOL)],
                         idx_v.at[par], sema)
        pltpu.async_copy(wgtt.at[pl.ds(q0, SBS), :], wgt_v.at[par], sema)

    def drain(par):
        pltpu.make_async_copy(idxt.at[pl.ds(0, SBS * NCOL)],
                              idx_v.at[par], sema).wait()
        pltpu.make_async_copy(wgtt.at[pl.ds(0, SBS), :],
                              wgt_v.at[par], sema).wait()

    def compute_sub(par, sub):
        rbuf = rows_v.at[sub % 2]
        wbuf = wgt_v.at[par]
        obuf = out_v

        def q_body(qq, carry2):
            rq = qq * NCOL
            row = sub * QBS + qq
            wvs = [wbuf[row, pl.ds(k * 16, 16)] for k in range(8)]
            for h in range(NH):
                woff = h // 4
                lbase = (h % 4) * 4
                acc = [jnp.zeros((16,), jnp.float32) for _ in range(4)]
                for c in range(4):
                    wv = wvs[c * 2 + woff]
                    rc = rq + c * 32 + h * 4
                    for p in range(4):
                        ws = _splat(wv, lbase + p)
                        lo = rbuf[rc + p, pl.ds(0, 16)]
                        hi = rbuf[rc + p, pl.ds(16, 16)]
                        k = 2 * (p % 2)
                        acc[k] = acc[k] + ws * lo
                        acc[k + 1] = acc[k + 1] + ws * hi
                obuf[row, pl.ds(h * 32, 16)] = acc[0] + acc[2]
                obuf[row, pl.ds(h * 32 + 16, 16)] = acc[1] + acc[3]
            return carry2

        lax.fori_loop(0, QBS, q_body, 0)

    def super_pair(i, carry):
        for par in (0, 1):
            sb = 2 * i + par
            q0 = base + sb * SBS
            drain(par)

            def fire(sub):
                return pltpu.async_copy(
                    vtab.at[idx_v.at[par].at[pl.ds(sub * QBS * NCOL,
                                                   QBS * NCOL)]],
                    rows_v.at[sub % 2],
                    sems[sub % 2],
                )

            handle = fire(0)
            nxt = sb + 1
            prefetch(jnp.where(nxt < nsb, nxt, 0), 1 - par)
            for sub in range(NSUB):
                nxt_h = fire(sub + 1) if sub + 1 < NSUB else None
                handle.wait()
                compute_sub(par, sub)
                handle = nxt_h

            pltpu.sync_copy(out_v, out.at[pl.ds(q0, SBS), :])
        return carry

    prefetch(0, 0)
    lax.fori_loop(0, QPW // (2 * SBS), super_pair, 0)
    drain(0)


_sc_gather = functools.partial(
    pl.kernel,
    out_type=jax.ShapeDtypeStruct((B * N, D), jnp.float32),
    mesh=plsc.VectorSubcoreMesh(core_axis_name="c", subcore_axis_name="s",
                                num_cores=2, num_subcores=16),
    compiler_params=pltpu.CompilerParams(needs_layout_passes=False,
                                         use_tc_tiling_on_sc=False),
    scratch_types=[
        pltpu.VMEM((2, SBS * NCOL), jnp.int32),
        pltpu.VMEM((2, SBS, NCOL), jnp.float32),
        pltpu.VMEM((2, QBS * NCOL, 32), jnp.float32),
        pltpu.VMEM((SBS, D), jnp.float32),
        pltpu.SemaphoreType.DMA,
        pltpu.SemaphoreType.DMA,
        pltpu.SemaphoreType.DMA,
        pltpu.SemaphoreType.DMA,
    ],
)(_sc_body)


def kernel(query, reference_points, value, spatial_shapes, level_start_index,
           W_value, b_value, W_off, b_off, W_attn, b_attn, W_out, b_out):
    wr = W_off.reshape(D, NH * NP, 2)
    wox = wr[..., 0]
    woy = wr[..., 1]
    br = b_off.reshape(NH * NP, 2)
    box = br[..., 0].reshape(1, NH * NP)
    boy = br[..., 1].reshape(1, NH * NP)
    gsum = jnp.asarray(np.kron(np.eye(NH, dtype=np.float32),
                               np.ones((NP, NP), np.float32)))

    vout, idxm, wgtm = _stage_a(
        query, reference_points, value,
        W_value, b_value.reshape(1, D),
        wox, box, woy, boy, W_attn, b_attn.reshape(1, NH * NP),
        gsum,
    )
    samp = _sc_gather(
        vout.reshape(B * N * NH, D // NH),
        idxm.reshape(B * N * NCOL),
        wgtm.reshape(B * N, NCOL),
    )
    return _stage_b(samp.reshape(B, N, D), W_out, b_out.reshape(1, D))


# speedup vs baseline: 1.3732x; 1.1138x over previous
import functools

import jax
import jax.numpy as jnp
import numpy as np
from jax import lax
from jax.experimental import pallas as pl
from jax.experimental.pallas import tpu as pltpu
from jax.experimental.pallas import tpu_sc as plsc

B = 4
N = 4096
D = 256
NH = 8
NP = 4
HG = 64
WG = 64
QBA = 512
QBS = 8
SBS = 64
NSUB = SBS // QBS
NWORK = 32
QPW = (B * N) // NWORK
NSLOT = NP * 4
NCOL = NH * NSLOT

_SPLAT_DNUMS = lax.GatherDimensionNumbers(
    offset_dims=(), collapsed_slice_dims=(0,), start_index_map=(0,))


def _splat(vec, s):
    return lax.gather(vec, jnp.full((16, 1), s, jnp.int32), _SPLAT_DNUMS, (1,),
                      mode=lax.GatherScatterMode.PROMISE_IN_BOUNDS)


def _stage_a_body(q_ref, rp_ref, val_ref, wv_ref, bv_ref, wox_ref, box_ref,
                  woy_ref, boy_ref, wat_ref, bat_ref, g_ref,
                  vout_ref, idx_ref, wgt_ref):
    q = q_ref[0]
    vout_ref[0] = (
        jnp.dot(val_ref[0], wv_ref[...], preferred_element_type=jnp.float32)
        + bv_ref[...]
    )
    offx = jnp.dot(q, wox_ref[...], preferred_element_type=jnp.float32) + box_ref[...]
    offy = jnp.dot(q, woy_ref[...], preferred_element_type=jnp.float32) + boy_ref[...]
    a = jnp.dot(q, wat_ref[...], preferred_element_type=jnp.float32) + bat_ref[...]
    m = jnp.max(a, axis=-1, keepdims=True)
    e = jnp.exp(a - m)
    denom = jnp.dot(e, g_ref[...], preferred_element_type=jnp.float32)
    attn = e / denom

    rp = rp_ref[0]
    x = rp[:, 0:1] * float(WG) - 0.5 + offx
    y = rp[:, 1:2] * float(HG) - 0.5 + offy
    x0 = jnp.floor(x)
    y0 = jnp.floor(y)
    lx = x - x0
    ly = y - y0
    x0i = x0.astype(jnp.int32)
    y0i = y0.astype(jnp.int32)
    hcol = lax.broadcasted_iota(jnp.int32, (QBA, NH * NP), 1) // NP
    bofs = pl.program_id(0) * (N * NH)
    wcorn = [(1.0 - lx) * (1.0 - ly), lx * (1.0 - ly),
             (1.0 - lx) * ly, lx * ly]

    idx_c = [None] * 4
    wgt_c = [None] * 4
    for c, (dx, dy) in enumerate(((0, 0), (1, 0), (0, 1), (1, 1))):
        cx = x0i + dx
        cy = y0i + dy
        valid = ((cx >= 0) & (cx < WG) & (cy >= 0) & (cy < HG))
        cell = jnp.clip(cy, 0, HG - 1) * WG + jnp.clip(cx, 0, WG - 1)
        idx_c[c] = bofs + cell * NH + hcol
        wgt_c[c] = wcorn[c] * attn * valid.astype(jnp.float32)

    idx_ref[0] = jnp.concatenate(idx_c, axis=1)
    wgt_ref[0] = jnp.concatenate(wgt_c, axis=1)


def _stage_a(query, ref_pts, value, wv, bv, wox, box, woy, boy, wat, bat, g):
    grid = (B, N // QBA)
    full = lambda shape: pl.BlockSpec(shape, lambda b, j: (0,) * len(shape))
    blk3 = lambda w: pl.BlockSpec((1, QBA, w), lambda b, j: (b, j, 0))
    return pl.pallas_call(
        _stage_a_body,
        grid=grid,
        in_specs=[
            blk3(D), blk3(2), blk3(D),
            full((D, D)), full((1, D)),
            full((D, 32)), full((1, 32)),
            full((D, 32)), full((1, 32)),
            full((D, 32)), full((1, 32)),
            full((32, 32)),
        ],
        out_specs=[blk3(D), blk3(NCOL), blk3(NCOL)],
        out_shape=[
            jax.ShapeDtypeStruct((B, N, D), jnp.float32),
            jax.ShapeDtypeStruct((B, N, NCOL), jnp.int32),
            jax.ShapeDtypeStruct((B, N, NCOL), jnp.float32),
        ],
    )(query, ref_pts, value, wv, bv, wox, box, woy, boy, wat, bat, g)


def _stage_b_body(s_ref, w_ref, b_ref, o_ref):
    o_ref[0] = (
        jnp.dot(s_ref[0], w_ref[...], preferred_element_type=jnp.float32)
        + b_ref[...]
    )


def _stage_b(samp, w_out, b_out2):
    return pl.pallas_call(
        _stage_b_body,
        grid=(B, N // QBA),
        in_specs=[
            pl.BlockSpec((1, QBA, D), lambda b, j: (b, j, 0)),
            pl.BlockSpec((D, D), lambda b, j: (0, 0)),
            pl.BlockSpec((1, D), lambda b, j: (0, 0)),
        ],
        out_specs=pl.BlockSpec((1, QBA, D), lambda b, j: (b, j, 0)),
        out_shape=jax.ShapeDtypeStruct((B, N, D), jnp.float32),
    )(samp, w_out, b_out2)


def _sc_body(vtab, idxt, wgtt, out, idx_v, wgt_v, rows_v, out_v, sem0, sem1,
             sema, semb):
    wid = lax.axis_index("s") * 2 + lax.axis_index("c")
    base = wid * QPW
    sems = (sem0, sem1)
    nsb = QPW // SBS

    def prefetch(sb, par):
        q0 = base + sb * SBS
        pltpu.async_copy(idxt.at[pl.ds(q0 * NCOL, SBS * NCOL)],
                         idx_v.at[par], sema)
        pltpu.async_copy(wgtt.at[pl.ds(q0, SBS), :], wgt_v.at[par], sema)

    def drain(par):
        pltpu.make_async_copy(idxt.at[pl.ds(0, SBS * NCOL)],
                              idx_v.at[par], sema).wait()
        pltpu.make_async_copy(wgtt.at[pl.ds(0, SBS), :],
                              wgt_v.at[par], sema).wait()

    def compute_sub(par, sub):
        rbuf = rows_v.at[sub % 2]
        wbuf = wgt_v.at[par]
        obuf = out_v

        def q_body(qq, carry2):
            rq = qq * NCOL
            row = sub * QBS + qq
            wvs = [wbuf[row, pl.ds(k * 16, 16)] for k in range(8)]
            for h in range(NH):
                woff = h // 4
                lbase = (h % 4) * 4
                acc = [jnp.zeros((16,), jnp.float32) for _ in range(4)]
                for c in range(4):
                    wv = wvs[c * 2 + woff]
                    rc = rq + c * 32 + h * 4
                    for p in range(4):
                        ws = _splat(wv, lbase + p)
                        lo = rbuf[rc + p, pl.ds(0, 16)]
                        hi = rbuf[rc + p, pl.ds(16, 16)]
                        k = 2 * (p % 2)
                        acc[k] = acc[k] + ws * lo
                        acc[k + 1] = acc[k + 1] + ws * hi
                obuf[row, pl.ds(h * 32, 16)] = acc[0] + acc[2]
                obuf[row, pl.ds(h * 32 + 16, 16)] = acc[1] + acc[3]
            return carry2

        lax.fori_loop(0, QBS, q_body, 0)

    def super_pair(i, carry):
        for par in (0, 1):
            sb = 2 * i + par
            q0 = base + sb * SBS
            drain(par)

            def fire(sub):
                return pltpu.async_copy(
                    vtab.at[idx_v.at[par].at[pl.ds(sub * QBS * NCOL,
                                                   QBS * NCOL)]],
                    rows_v.at[sub % 2],
                    sems[sub % 2],
                )

            handle = fire(0)
            nxt = sb + 1
            prefetch(jnp.where(nxt < nsb, nxt, 0), 1 - par)
            for sub in range(NSUB):
                nxt_h = fire(sub + 1) if sub + 1 < NSUB else None
                handle.wait()
                compute_sub(par, sub)
                handle = nxt_h

            pltpu.sync_copy(out_v, out.at[pl.ds(q0, SBS), :])
        return carry

    prefetch(0, 0)
    lax.fori_loop(0, QPW // (2 * SBS), super_pair, 0)
    drain(0)


_sc_gather = functools.partial(
    pl.kernel,
    out_type=jax.ShapeDtypeStruct((B * N, D), jnp.float32),
    mesh=plsc.VectorSubcoreMesh(core_axis_name="c", subcore_axis_name="s",
                                num_cores=2, num_subcores=16),
    compiler_params=pltpu.CompilerParams(needs_layout_passes=False,
                                         use_tc_tiling_on_sc=False),
    scratch_types=[
        pltpu.VMEM((2, SBS * NCOL), jnp.int32),
        pltpu.VMEM((2, SBS, NCOL), jnp.float32),
        pltpu.VMEM((2, QBS * NCOL, 32), jnp.float32),
        pltpu.VMEM((SBS, D), jnp.float32),
        pltpu.SemaphoreType.DMA,
        pltpu.SemaphoreType.DMA,
        pltpu.SemaphoreType.DMA,
        pltpu.SemaphoreType.DMA,
    ],
)(_sc_body)


def kernel(query, reference_points, value, spatial_shapes, level_start_index,
           W_value, b_value, W_off, b_off, W_attn, b_attn, W_out, b_out):
    wr = W_off.reshape(D, NH * NP, 2)
    wox = wr[..., 0]
    woy = wr[..., 1]
    br = b_off.reshape(NH * NP, 2)
    box = br[..., 0].reshape(1, NH * NP)
    boy = br[..., 1].reshape(1, NH * NP)
    gsum = jnp.asarray(np.kron(np.eye(NH, dtype=np.float32),
                               np.ones((NP, NP), np.float32)))

    vout, idxm, wgtm = _stage_a(
        query, reference_points, value,
        W_value, b_value.reshape(1, D),
        wox, box, woy, boy, W_attn, b_attn.reshape(1, NH * NP),
        gsum,
    )
    samp = _sc_gather(
        vout.reshape(B * N * NH, D // NH),
        idxm.reshape(B * N * NCOL),
        wgtm.reshape(B * N, NCOL),
    )
    return _stage_b(samp.reshape(B, N, D), W_out, b_out.reshape(1, D))


# speedup vs baseline: 1.3964x; 1.0169x over previous
import functools

import jax
import jax.numpy as jnp
import numpy as np
from jax import lax
from jax.experimental import pallas as pl
from jax.experimental.pallas import tpu as pltpu
from jax.experimental.pallas import tpu_sc as plsc

B = 4
N = 4096
D = 256
NH = 8
NP = 4
HG = 64
WG = 64
QBA = 1024
QBS = 8
SBS = 64
NSUB = SBS // QBS
NWORK = 32
QPW = (B * N) // NWORK
NSLOT = NP * 4
NCOL = NH * NSLOT

_SPLAT_DNUMS = lax.GatherDimensionNumbers(
    offset_dims=(), collapsed_slice_dims=(0,), start_index_map=(0,))


def _splat(vec, s):
    return lax.gather(vec, jnp.full((16, 1), s, jnp.int32), _SPLAT_DNUMS, (1,),
                      mode=lax.GatherScatterMode.PROMISE_IN_BOUNDS)


def _stage_a_body(q_ref, rp_ref, val_ref, wv_ref, bv_ref, wox_ref, box_ref,
                  woy_ref, boy_ref, wat_ref, bat_ref, g_ref,
                  vout_ref, idx_ref, wgt_ref):
    q = q_ref[0]
    vout_ref[0] = (
        jnp.dot(val_ref[0], wv_ref[...], preferred_element_type=jnp.float32)
        + bv_ref[...]
    )
    offx = jnp.dot(q, wox_ref[...], preferred_element_type=jnp.float32) + box_ref[...]
    offy = jnp.dot(q, woy_ref[...], preferred_element_type=jnp.float32) + boy_ref[...]
    a = jnp.dot(q, wat_ref[...], preferred_element_type=jnp.float32) + bat_ref[...]
    m = jnp.max(a, axis=-1, keepdims=True)
    e = jnp.exp(a - m)
    denom = jnp.dot(e, g_ref[...], preferred_element_type=jnp.float32)
    attn = e / denom

    rp = rp_ref[0]
    x = rp[:, 0:1] * float(WG) - 0.5 + offx
    y = rp[:, 1:2] * float(HG) - 0.5 + offy
    x0 = jnp.floor(x)
    y0 = jnp.floor(y)
    lx = x - x0
    ly = y - y0
    x0i = x0.astype(jnp.int32)
    y0i = y0.astype(jnp.int32)
    hcol = lax.broadcasted_iota(jnp.int32, (QBA, NH * NP), 1) // NP
    bofs = pl.program_id(0) * (N * NH)
    wcorn = [(1.0 - lx) * (1.0 - ly), lx * (1.0 - ly),
             (1.0 - lx) * ly, lx * ly]

    idx_c = [None] * 4
    wgt_c = [None] * 4
    for c, (dx, dy) in enumerate(((0, 0), (1, 0), (0, 1), (1, 1))):
        cx = x0i + dx
        cy = y0i + dy
        valid = ((cx >= 0) & (cx < WG) & (cy >= 0) & (cy < HG))
        cell = jnp.clip(cy, 0, HG - 1) * WG + jnp.clip(cx, 0, WG - 1)
        idx_c[c] = bofs + cell * NH + hcol
        wgt_c[c] = wcorn[c] * attn * valid.astype(jnp.float32)

    idx_ref[0] = jnp.concatenate(idx_c, axis=1)
    wgt_ref[0] = jnp.concatenate(wgt_c, axis=1)


def _stage_a(query, ref_pts, value, wv, bv, wox, box, woy, boy, wat, bat, g):
    grid = (B, N // QBA)
    full = lambda shape: pl.BlockSpec(shape, lambda b, j: (0,) * len(shape))
    blk3 = lambda w: pl.BlockSpec((1, QBA, w), lambda b, j: (b, j, 0))
    return pl.pallas_call(
        _stage_a_body,
        grid=grid,
        in_specs=[
            blk3(D), blk3(2), blk3(D),
            full((D, D)), full((1, D)),
            full((D, 32)), full((1, 32)),
            full((D, 32)), full((1, 32)),
            full((D, 32)), full((1, 32)),
            full((32, 32)),
        ],
        out_specs=[blk3(D), blk3(NCOL), blk3(NCOL)],
        out_shape=[
            jax.ShapeDtypeStruct((B, N, D), jnp.float32),
            jax.ShapeDtypeStruct((B, N, NCOL), jnp.int32),
            jax.ShapeDtypeStruct((B, N, NCOL), jnp.float32),
        ],
    )(query, ref_pts, value, wv, bv, wox, box, woy, boy, wat, bat, g)


def _stage_b_body(s_ref, w_ref, b_ref, o_ref):
    o_ref[0] = (
        jnp.dot(s_ref[0], w_ref[...], preferred_element_type=jnp.float32)
        + b_ref[...]
    )


def _stage_b(samp, w_out, b_out2):
    return pl.pallas_call(
        _stage_b_body,
        grid=(B, N // QBA),
        in_specs=[
            pl.BlockSpec((1, QBA, D), lambda b, j: (b, j, 0)),
            pl.BlockSpec((D, D), lambda b, j: (0, 0)),
            pl.BlockSpec((1, D), lambda b, j: (0, 0)),
        ],
        out_specs=pl.BlockSpec((1, QBA, D), lambda b, j: (b, j, 0)),
        out_shape=jax.ShapeDtypeStruct((B, N, D), jnp.float32),
    )(samp, w_out, b_out2)


def _sc_body(vtab, idxt, wgtt, out, idx_v, wgt_v, rows_v, out_v, sem0, sem1,
             sema, semb):
    wid = lax.axis_index("s") * 2 + lax.axis_index("c")
    base = wid * QPW
    sems = (sem0, sem1)
    nsb = QPW // SBS

    def prefetch(sb, par):
        q0 = base + sb * SBS
        pltpu.async_copy(idxt.at[pl.ds(q0 * NCOL, SBS * NCOL)],
                         idx_v.at[par], sema)
        pltpu.async_copy(wgtt.at[pl.ds(q0, SBS), :], wgt_v.at[par], sema)

    def drain(par):
        pltpu.make_async_copy(idxt.at[pl.ds(0, SBS * NCOL)],
                              idx_v.at[par], sema).wait()
        pltpu.make_async_copy(wgtt.at[pl.ds(0, SBS), :],
                              wgt_v.at[par], sema).wait()

    def compute_sub(par, sub):
        rbuf = rows_v.at[sub % 2]
        wbuf = wgt_v.at[par]
        obuf = out_v

        def q_body(qq, carry2):
            rq = qq * NCOL
            row = sub * QBS + qq
            wvs = [wbuf[row, pl.ds(k * 16, 16)] for k in range(8)]
            for h in range(NH):
                woff = h // 4
                lbase = (h % 4) * 4
                acc = [jnp.zeros((16,), jnp.float32) for _ in range(4)]
                for c in range(4):
                    wv = wvs[c * 2 + woff]
                    rc = rq + c * 32 + h * 4
                    for p in range(4):
                        ws = _splat(wv, lbase + p)
                        lo = rbuf[rc + p, pl.ds(0, 16)]
                        hi = rbuf[rc + p, pl.ds(16, 16)]
                        k = 2 * (p % 2)
                        acc[k] = acc[k] + ws * lo
                        acc[k + 1] = acc[k + 1] + ws * hi
                obuf[row, pl.ds(h * 32, 16)] = acc[0] + acc[2]
                obuf[row, pl.ds(h * 32 + 16, 16)] = acc[1] + acc[3]
            return carry2

        lax.fori_loop(0, QBS, q_body, 0)

    def super_pair(i, carry):
        for par in (0, 1):
            sb = 2 * i + par
            q0 = base + sb * SBS
            drain(par)

            def fire(sub):
                return pltpu.async_copy(
                    vtab.at[idx_v.at[par].at[pl.ds(sub * QBS * NCOL,
                                                   QBS * NCOL)]],
                    rows_v.at[sub % 2],
                    sems[sub % 2],
                )

            handle = fire(0)
            nxt = sb + 1
            prefetch(jnp.where(nxt < nsb, nxt, 0), 1 - par)
            for sub in range(NSUB):
                nxt_h = fire(sub + 1) if sub + 1 < NSUB else None
                handle.wait()
                compute_sub(par, sub)
                handle = nxt_h

            pltpu.sync_copy(out_v, out.at[pl.ds(q0, SBS), :])
        return carry

    prefetch(0, 0)
    lax.fori_loop(0, QPW // (2 * SBS), super_pair, 0)
    drain(0)


_sc_gather = functools.partial(
    pl.kernel,
    out_type=jax.ShapeDtypeStruct((B * N, D), jnp.float32),
    mesh=plsc.VectorSubcoreMesh(core_axis_name="c", subcore_axis_name="s",
                                num_cores=2, num_subcores=16),
    compiler_params=pltpu.CompilerParams(needs_layout_passes=False,
                                         use_tc_tiling_on_sc=False),
    scratch_types=[
        pltpu.VMEM((2, SBS * NCOL), jnp.int32),
        pltpu.VMEM((2, SBS, NCOL), jnp.float32),
        pltpu.VMEM((2, QBS * NCOL, 32), jnp.float32),
        pltpu.VMEM((SBS, D), jnp.float32),
        pltpu.SemaphoreType.DMA,
        pltpu.SemaphoreType.DMA,
        pltpu.SemaphoreType.DMA,
        pltpu.SemaphoreType.DMA,
    ],
)(_sc_body)


def kernel(query, reference_points, value, spatial_shapes, level_start_index,
           W_value, b_value, W_off, b_off, W_attn, b_attn, W_out, b_out):
    wr = W_off.reshape(D, NH * NP, 2)
    wox = wr[..., 0]
    woy = wr[..., 1]
    br = b_off.reshape(NH * NP, 2)
    box = br[..., 0].reshape(1, NH * NP)
    boy = br[..., 1].reshape(1, NH * NP)
    gsum = jnp.asarray(np.kron(np.eye(NH, dtype=np.float32),
                               np.ones((NP, NP), np.float32)))

    vout, idxm, wgtm = _stage_a(
        query, reference_points, value,
        W_value, b_value.reshape(1, D),
        wox, box, woy, boy, W_attn, b_attn.reshape(1, NH * NP),
        gsum,
    )
    samp = _sc_gather(
        vout.reshape(B * N * NH, D // NH),
        idxm.reshape(B * N * NCOL),
        wgtm.reshape(B * N, NCOL),
    )
    return _stage_b(samp.reshape(B, N, D), W_out, b_out.reshape(1, D))


# speedup vs baseline: 1.4238x; 1.0196x over previous
import functools

import jax
import jax.numpy as jnp
import numpy as np
from jax import lax
from jax.experimental import pallas as pl
from jax.experimental.pallas import tpu as pltpu
from jax.experimental.pallas import tpu_sc as plsc

B = 4
N = 4096
D = 256
NH = 8
NP = 4
HG = 64
WG = 64
QBA = 2048
QBS = 8
SBS = 64
NSUB = SBS // QBS
NWORK = 32
QPW = (B * N) // NWORK
NSLOT = NP * 4
NCOL = NH * NSLOT

_SPLAT_DNUMS = lax.GatherDimensionNumbers(
    offset_dims=(), collapsed_slice_dims=(0,), start_index_map=(0,))


def _splat(vec, s):
    return lax.gather(vec, jnp.full((16, 1), s, jnp.int32), _SPLAT_DNUMS, (1,),
                      mode=lax.GatherScatterMode.PROMISE_IN_BOUNDS)


def _stage_a_body(q_ref, rp_ref, val_ref, wv_ref, bv_ref, wox_ref, box_ref,
                  woy_ref, boy_ref, wat_ref, bat_ref, g_ref,
                  vout_ref, idx_ref, wgt_ref):
    q = q_ref[0]
    vout_ref[0] = (
        jnp.dot(val_ref[0], wv_ref[...], preferred_element_type=jnp.float32)
        + bv_ref[...]
    )
    offx = jnp.dot(q, wox_ref[...], preferred_element_type=jnp.float32) + box_ref[...]
    offy = jnp.dot(q, woy_ref[...], preferred_element_type=jnp.float32) + boy_ref[...]
    a = jnp.dot(q, wat_ref[...], preferred_element_type=jnp.float32) + bat_ref[...]
    m = jnp.max(a, axis=-1, keepdims=True)
    e = jnp.exp(a - m)
    denom = jnp.dot(e, g_ref[...], preferred_element_type=jnp.float32)
    attn = e / denom

    rp = rp_ref[0]
    x = rp[:, 0:1] * float(WG) - 0.5 + offx
    y = rp[:, 1:2] * float(HG) - 0.5 + offy
    x0 = jnp.floor(x)
    y0 = jnp.floor(y)
    lx = x - x0
    ly = y - y0
    x0i = x0.astype(jnp.int32)
    y0i = y0.astype(jnp.int32)
    hcol = lax.broadcasted_iota(jnp.int32, (QBA, NH * NP), 1) // NP
    bofs = pl.program_id(0) * (N * NH)
    wcorn = [(1.0 - lx) * (1.0 - ly), lx * (1.0 - ly),
             (1.0 - lx) * ly, lx * ly]

    idx_c = [None] * 4
    wgt_c = [None] * 4
    for c, (dx, dy) in enumerate(((0, 0), (1, 0), (0, 1), (1, 1))):
        cx = x0i + dx
        cy = y0i + dy
        valid = ((cx >= 0) & (cx < WG) & (cy >= 0) & (cy < HG))
        cell = jnp.clip(cy, 0, HG - 1) * WG + jnp.clip(cx, 0, WG - 1)
        idx_c[c] = bofs + cell * NH + hcol
        wgt_c[c] = wcorn[c] * attn * valid.astype(jnp.float32)

    idx_ref[0] = jnp.concatenate(idx_c, axis=1)
    wgt_ref[0] = jnp.concatenate(wgt_c, axis=1)


def _stage_a(query, ref_pts, value, wv, bv, wox, box, woy, boy, wat, bat, g):
    grid = (B, N // QBA)
    full = lambda shape: pl.BlockSpec(shape, lambda b, j: (0,) * len(shape))
    blk3 = lambda w: pl.BlockSpec((1, QBA, w), lambda b, j: (b, j, 0))
    return pl.pallas_call(
        _stage_a_body,
        grid=grid,
        in_specs=[
            blk3(D), blk3(2), blk3(D),
            full((D, D)), full((1, D)),
            full((D, 32)), full((1, 32)),
            full((D, 32)), full((1, 32)),
            full((D, 32)), full((1, 32)),
            full((32, 32)),
        ],
        out_specs=[blk3(D), blk3(NCOL), blk3(NCOL)],
        out_shape=[
            jax.ShapeDtypeStruct((B, N, D), jnp.float32),
            jax.ShapeDtypeStruct((B, N, NCOL), jnp.int32),
            jax.ShapeDtypeStruct((B, N, NCOL), jnp.float32),
        ],
    )(query, ref_pts, value, wv, bv, wox, box, woy, boy, wat, bat, g)


def _stage_b_body(s_ref, w_ref, b_ref, o_ref):
    o_ref[0] = (
        jnp.dot(s_ref[0], w_ref[...], preferred_element_type=jnp.float32)
        + b_ref[...]
    )


def _stage_b(samp, w_out, b_out2):
    return pl.pallas_call(
        _stage_b_body,
        grid=(B, N // QBA),
        in_specs=[
            pl.BlockSpec((1, QBA, D), lambda b, j: (b, j, 0)),
            pl.BlockSpec((D, D), lambda b, j: (0, 0)),
            pl.BlockSpec((1, D), lambda b, j: (0, 0)),
        ],
        out_specs=pl.BlockSpec((1, QBA, D), lambda b, j: (b, j, 0)),
        out_shape=jax.ShapeDtypeStruct((B, N, D), jnp.float32),
    )(samp, w_out, b_out2)


def _sc_body(vtab, idxt, wgtt, out, idx_v, wgt_v, rows_v, out_v, sem0, sem1,
             sema, semb):
    wid = lax.axis_index("s") * 2 + lax.axis_index("c")
    base = wid * QPW
    sems = (sem0, sem1)
    nsb = QPW // SBS

    def prefetch(sb, par):
        q0 = base + sb * SBS
        pltpu.async_copy(idxt.at[pl.ds(q0 * NCOL, SBS * NCOL)],
                         idx_v.at[par], sema)
        pltpu.async_copy(wgtt.at[pl.ds(q0, SBS), :], wgt_v.at[par], sema)

    def drain(par):
        pltpu.make_async_copy(idxt.at[pl.ds(0, SBS * NCOL)],
                              idx_v.at[par], sema).wait()
        pltpu.make_async_copy(wgtt.at[pl.ds(0, SBS), :],
                              wgt_v.at[par], sema).wait()

    def compute_sub(par, sub):
        rbuf = rows_v.at[sub % 2]
        wbuf = wgt_v.at[par]
        obuf = out_v

        def q_body(qq, carry2):
            rq = qq * NCOL
            row = sub * QBS + qq
            wvs = [wbuf[row, pl.ds(k * 16, 16)] for k in range(8)]
            for h in range(NH):
                woff = h // 4
                lbase = (h % 4) * 4
                acc = [jnp.zeros((16,), jnp.float32) for _ in range(4)]
                for c in range(4):
                    wv = wvs[c * 2 + woff]
                    rc = rq + c * 32 + h * 4
                    for p in range(4):
                        ws = _splat(wv, lbase + p)
                        lo = rbuf[rc + p, pl.ds(0, 16)]
                        hi = rbuf[rc + p, pl.ds(16, 16)]
                        k = 2 * (p % 2)
                        acc[k] = acc[k] + ws * lo
                        acc[k + 1] = acc[k + 1] + ws * hi
                obuf[row, pl.ds(h * 32, 16)] = acc[0] + acc[2]
                obuf[row, pl.ds(h * 32 + 16, 16)] = acc[1] + acc[3]
            return carry2

        lax.fori_loop(0, QBS, q_body, 0)

    def super_pair(i, carry):
        for par in (0, 1):
            sb = 2 * i + par
            q0 = base + sb * SBS
            drain(par)

            def fire(sub):
                return pltpu.async_copy(
                    vtab.at[idx_v.at[par].at[pl.ds(sub * QBS * NCOL,
                                                   QBS * NCOL)]],
                    rows_v.at[sub % 2],
                    sems[sub % 2],
                )

            handle = fire(0)
            nxt = sb + 1
            prefetch(jnp.where(nxt < nsb, nxt, 0), 1 - par)
            for sub in range(NSUB):
                nxt_h = fire(sub + 1) if sub + 1 < NSUB else None
                handle.wait()
                compute_sub(par, sub)
                handle = nxt_h

            pltpu.sync_copy(out_v, out.at[pl.ds(q0, SBS), :])
        return carry

    prefetch(0, 0)
    lax.fori_loop(0, QPW // (2 * SBS), super_pair, 0)
    drain(0)


_sc_gather = functools.partial(
    pl.kernel,
    out_type=jax.ShapeDtypeStruct((B * N, D), jnp.float32),
    mesh=plsc.VectorSubcoreMesh(core_axis_name="c", subcore_axis_name="s",
                                num_cores=2, num_subcores=16),
    compiler_params=pltpu.CompilerParams(needs_layout_passes=False,
                                         use_tc_tiling_on_sc=False),
    scratch_types=[
        pltpu.VMEM((2, SBS * NCOL), jnp.int32),
        pltpu.VMEM((2, SBS, NCOL), jnp.float32),
        pltpu.VMEM((2, QBS * NCOL, 32), jnp.float32),
        pltpu.VMEM((SBS, D), jnp.float32),
        pltpu.SemaphoreType.DMA,
        pltpu.SemaphoreType.DMA,
        pltpu.SemaphoreType.DMA,
        pltpu.SemaphoreType.DMA,
    ],
)(_sc_body)


def kernel(query, reference_points, value, spatial_shapes, level_start_index,
           W_value, b_value, W_off, b_off, W_attn, b_attn, W_out, b_out):
    wr = W_off.reshape(D, NH * NP, 2)
    wox = wr[..., 0]
    woy = wr[..., 1]
    br = b_off.reshape(NH * NP, 2)
    box = br[..., 0].reshape(1, NH * NP)
    boy = br[..., 1].reshape(1, NH * NP)
    gsum = jnp.asarray(np.kron(np.eye(NH, dtype=np.float32),
                               np.ones((NP, NP), np.float32)))

    vout, idxm, wgtm = _stage_a(
        query, reference_points, value,
        W_value, b_value.reshape(1, D),
        wox, box, woy, boy, W_attn, b_attn.reshape(1, NH * NP),
        gsum,
    )
    samp = _sc_gather(
        vout.reshape(B * N * NH, D // NH),
        idxm.reshape(B * N * NCOL),
        wgtm.reshape(B * N, NCOL),
    )
    return _stage_b(samp.reshape(B, N, D), W_out, b_out.reshape(1, D))
